# Initial kernel scaffold; baseline (speedup 1.0000x reference)
import jax
import jax.numpy as jnp
from jax.experimental import pallas as pl


def kernel(indices, table):
    raise NotImplementedError("write your pallas kernel here")


# speedup vs baseline: 1.7954x; 1.7954x over previous
import functools

import jax
import jax.numpy as jnp
from jax import lax
from jax.experimental import pallas as pl
from jax.experimental.pallas import tpu as pltpu
from jax.experimental.pallas import tpu_sc as plsc

EMBED_DIM = 64
NUM_WORKERS = 32
CHUNK = 512


def _sc_gather(idx_flat, table):
    n = idx_flat.shape[0]
    per_worker = n // NUM_WORKERS
    n_chunks = per_worker // CHUNK
    mesh = plsc.VectorSubcoreMesh(core_axis_name="c", subcore_axis_name="s")

    @functools.partial(
        pl.kernel,
        mesh=mesh,
        out_type=jax.ShapeDtypeStruct((n, EMBED_DIM), jnp.float32),
        scratch_types=[
            pltpu.VMEM((CHUNK,), jnp.int32),
            pltpu.VMEM((CHUNK, EMBED_DIM), jnp.float32),
            pltpu.SemaphoreType.DMA,
        ],
        compiler_params=pltpu.CompilerParams(use_tc_tiling_on_sc=False),
    )
    def grab(idx_hbm, table_hbm, out_hbm, idx_v, rows_v, sem):
        wid = lax.axis_index("s") * 2 + lax.axis_index("c")
        base = wid * per_worker

        def body(i, carry):
            off = base + i * CHUNK
            pltpu.sync_copy(idx_hbm.at[pl.ds(off, CHUNK)], idx_v)
            pltpu.async_copy(table_hbm.at[idx_v], rows_v, sem).wait()
            pltpu.sync_copy(rows_v, out_hbm.at[pl.ds(off, CHUNK)])
            return carry

        lax.fori_loop(0, n_chunks, body, 0)

    return grab(idx_flat, table)


def kernel(indices, table):
    b, h = indices.shape
    idx_flat = indices.reshape(-1).astype(jnp.int32)
    out = _sc_gather(idx_flat, table)
    return out.reshape(b, h, EMBED_DIM)


# speedup vs baseline: 1.8863x; 1.0506x over previous
import functools

import jax
import jax.numpy as jnp
from jax import lax
from jax.experimental import pallas as pl
from jax.experimental.pallas import tpu as pltpu
from jax.experimental.pallas import tpu_sc as plsc

EMBED_DIM = 64
NUM_WORKERS = 32
CHUNK = 640


def _sc_gather(idx_flat, table):
    n = idx_flat.shape[0]
    per_worker = n // NUM_WORKERS
    n_chunks = per_worker // CHUNK
    assert n_chunks % 2 == 0
    mesh = plsc.VectorSubcoreMesh(core_axis_name="c", subcore_axis_name="s")

    @functools.partial(
        pl.kernel,
        mesh=mesh,
        out_type=jax.ShapeDtypeStruct((n, EMBED_DIM), jnp.float32),
        scratch_types=[
            pltpu.VMEM((per_worker,), jnp.int32),
            pltpu.VMEM((CHUNK, EMBED_DIM), jnp.float32),
            pltpu.VMEM((CHUNK, EMBED_DIM), jnp.float32),
            pltpu.SemaphoreType.DMA,
            pltpu.SemaphoreType.DMA,
            pltpu.SemaphoreType.DMA,
            pltpu.SemaphoreType.DMA,
        ],
        compiler_params=pltpu.CompilerParams(use_tc_tiling_on_sc=False),
    )
    def grab(idx_hbm, table_hbm, out_hbm, idx_v, rows0, rows1, g0, g1, o0, o1):
        wid = lax.axis_index("s") * 2 + lax.axis_index("c")
        base = wid * per_worker
        pltpu.sync_copy(idx_hbm.at[pl.ds(base, per_worker)], idx_v)

        def gather(j, rows, sem):
            pltpu.async_copy(
                table_hbm.at[idx_v.at[pl.ds(j * CHUNK, CHUNK)]], rows, sem)

        def put(j, rows, sem):
            pltpu.async_copy(rows, out_hbm.at[pl.ds(base + j * CHUNK, CHUNK)], sem)

        def wait_gather(rows, sem):
            pltpu.make_async_copy(table_hbm.at[pl.ds(0, CHUNK)], rows, sem).wait()

        def wait_put(rows, sem):
            pltpu.make_async_copy(rows, out_hbm.at[pl.ds(base, CHUNK)], sem).wait()

        gather(0, rows0, g0)
        gather(1, rows1, g1)

        def body(i, carry):
            j0 = 2 * i
            wait_gather(rows0, g0)
            put(j0, rows0, o0)

            @pl.when(i + 1 < n_chunks // 2)
            def _():
                wait_put(rows0, o0)
                gather(j0 + 2, rows0, g0)

            wait_gather(rows1, g1)
            put(j0 + 1, rows1, o1)

            @pl.when(i + 1 < n_chunks // 2)
            def _():
                wait_put(rows1, o1)
                gather(j0 + 3, rows1, g1)

            return carry

        lax.fori_loop(0, n_chunks // 2, body, 0)
        wait_put(rows0, o0)
        wait_put(rows1, o1)

    return grab(idx_flat, table)


def kernel(indices, table):
    b, h = indices.shape
    idx_flat = indices.reshape(-1).astype(jnp.int32)
    out = _sc_gather(idx_flat, table)
    return out.reshape(b, h, EMBED_DIM)
